# baseline (device time: 277607 ns/iter reference)
import jax
import jax.numpy as jnp
from jax import lax
from jax.experimental import pallas as pl
from jax.experimental.pallas import tpu as pltpu

B = 4
S = 1024
H_SHARD = 16
D = 128
K = H_SHARD * D
N = 4096
S_HALF = S // 2
NT = 2048


def kernel(O, Wo):
    o = O.reshape(B, S, K).astype(jnp.bfloat16)
    w = Wo.astype(jnp.bfloat16)

    def body(
        o_hbm, w_ref, out_hbm,
        o_send, o_keep, send_buf, recv_buf, out_stage,
        load_sem, keep_sem, store_sems, send_sems, recv_sems, credit_sem,
    ):
        my_x = lax.axis_index("x")
        my_y = lax.axis_index("y")
        my_z = lax.axis_index("z")
        partner = (my_x, my_y, 1 - my_z)

        barrier_sem = pltpu.get_barrier_semaphore()
        pl.semaphore_signal(
            barrier_sem, inc=1,
            device_id=partner, device_id_type=pl.DeviceIdType.MESH,
        )
        pl.semaphore_wait(barrier_sem, 1)

        send_off = (1 - my_z) * S_HALF
        keep_off = my_z * S_HALF

        pending_store = [None, None]

        def consume(b, rdma_b):
            slot = b % 2
            for n in range(N // NT):
                s = n % 2
                if pending_store[s] is not None:
                    pending_store[s].wait()
                out_stage[s] = jnp.dot(
                    o_keep[...], w_ref[:, n * NT:(n + 1) * NT],
                    preferred_element_type=jnp.float32,
                )
                if n == 0:
                    rdma_b.wait_recv()
                out_stage[s] = out_stage[s] + recv_buf[
                    slot, :, n * NT:(n + 1) * NT
                ].astype(jnp.float32)
                st = pltpu.make_async_copy(
                    out_stage.at[s],
                    out_hbm.at[b, :, pl.ds(n * NT, NT)],
                    store_sems.at[s],
                )
                st.start()
                pending_store[s] = st
            pl.semaphore_signal(
                credit_sem, inc=1,
                device_id=partner, device_id_type=pl.DeviceIdType.MESH,
            )

        rdmas = []
        for b in range(B):
            slot = b % 2

            cp_s = pltpu.make_async_copy(
                o_hbm.at[b, pl.ds(send_off, S_HALF), :],
                o_send, load_sem,
            )
            cp_s.start()
            if b >= 1:
                cp_k = pltpu.make_async_copy(
                    o_hbm.at[b - 1, pl.ds(keep_off, S_HALF), :],
                    o_keep, keep_sem,
                )
                cp_k.start()
            cp_s.wait()
            if b >= 2:
                pl.semaphore_wait(credit_sem, 1)
                rdmas[b - 2].wait_send()
            for n in range(N // NT):
                send_buf[slot, :, n * NT:(n + 1) * NT] = jnp.dot(
                    o_send[...], w_ref[:, n * NT:(n + 1) * NT],
                    preferred_element_type=jnp.float32,
                ).astype(jnp.bfloat16)
            rdma = pltpu.make_async_remote_copy(
                src_ref=send_buf.at[slot],
                dst_ref=recv_buf.at[slot],
                send_sem=send_sems.at[slot],
                recv_sem=recv_sems.at[slot],
                device_id=partner,
                device_id_type=pl.DeviceIdType.MESH,
            )
            rdma.start()
            rdmas.append(rdma)

            if b >= 1:
                cp_k.wait()
                consume(b - 1, rdmas[b - 1])

        cp_k = pltpu.make_async_copy(
            o_hbm.at[B - 1, pl.ds(keep_off, S_HALF), :],
            o_keep, keep_sem,
        )
        cp_k.start()
        cp_k.wait()
        consume(B - 1, rdmas[B - 1])

        rdmas[B - 2].wait_send()
        rdmas[B - 1].wait_send()
        pl.semaphore_wait(credit_sem, 2)
        for st in pending_store:
            if st is not None:
                st.wait()

    return pl.pallas_call(
        body,
        out_shape=jax.ShapeDtypeStruct((B, S_HALF, N), jnp.float32),
        in_specs=[
            pl.BlockSpec(memory_space=pl.ANY),
            pl.BlockSpec(memory_space=pltpu.VMEM),
        ],
        out_specs=pl.BlockSpec(memory_space=pl.ANY),
        scratch_shapes=[
            pltpu.VMEM((S_HALF, K), jnp.bfloat16),
            pltpu.VMEM((S_HALF, K), jnp.bfloat16),
            pltpu.VMEM((2, S_HALF, N), jnp.bfloat16),
            pltpu.VMEM((2, S_HALF, N), jnp.bfloat16),
            pltpu.VMEM((2, S_HALF, NT), jnp.float32),
            pltpu.SemaphoreType.DMA,
            pltpu.SemaphoreType.DMA,
            pltpu.SemaphoreType.DMA((2,)),
            pltpu.SemaphoreType.DMA((2,)),
            pltpu.SemaphoreType.DMA((2,)),
            pltpu.SemaphoreType.REGULAR,
        ],
        compiler_params=pltpu.CompilerParams(
            collective_id=0,
            vmem_limit_bytes=50250000,
        ),
    )(o, w)


# device time: 269434 ns/iter; 1.0303x vs baseline; 1.0303x over previous
import jax
import jax.numpy as jnp
from jax import lax
from jax.experimental import pallas as pl
from jax.experimental.pallas import tpu as pltpu

B = 4
S = 1024
H_SHARD = 16
D = 128
K = H_SHARD * D
N = 4096
S_HALF = S // 2
NT = 2048


def kernel(O, Wo):
    o = O.astype(jnp.bfloat16).reshape(B, S, K)
    w = Wo.astype(jnp.bfloat16)

    def body(
        o_hbm, w_ref, out_hbm,
        o_send, o_keep, send_buf, recv_buf, acc_stage, out_stage,
        load_sem, keep_sem, store_sems, send_sems, recv_sems, credit_sem,
    ):
        my_x = lax.axis_index("x")
        my_y = lax.axis_index("y")
        my_z = lax.axis_index("z")
        partner = (my_x, my_y, 1 - my_z)

        barrier_sem = pltpu.get_barrier_semaphore()
        pl.semaphore_signal(
            barrier_sem, inc=1,
            device_id=partner, device_id_type=pl.DeviceIdType.MESH,
        )
        pl.semaphore_wait(barrier_sem, 1)

        send_off = (1 - my_z) * S_HALF
        keep_off = my_z * S_HALF

        def start_row_load(b, off, dst, sem):
            cp = pltpu.make_async_copy(
                o_hbm.at[b, pl.ds(off, S_HALF), :], dst, sem
            )
            cp.start()
            return cp

        pending_store = [None, None]

        def consume(b, rdma_b):
            slot = b % 2
            for n in range(N // NT):
                s = n % 2
                acc_stage[...] = jnp.dot(
                    o_keep[...], w_ref[:, n * NT:(n + 1) * NT],
                    preferred_element_type=jnp.float32,
                )
                if n == 0:
                    rdma_b.wait_recv()
                acc_stage[...] = acc_stage[...] + recv_buf[
                    slot, :, n * NT:(n + 1) * NT
                ].astype(jnp.float32)
                if pending_store[0] is not None:
                    pending_store[0].wait()
                out_stage[...] = acc_stage[...].astype(jnp.bfloat16)
                st = pltpu.make_async_copy(
                    out_stage,
                    out_hbm.at[b, :, pl.ds(n * NT, NT)],
                    store_sems.at[s],
                )
                st.start()
                pending_store[0] = st
            pl.semaphore_signal(
                credit_sem, inc=1,
                device_id=partner, device_id_type=pl.DeviceIdType.MESH,
            )

        rdmas = []
        for b in range(B):
            slot = b % 2

            cps_s = start_row_load(b, send_off, o_send, load_sem)
            if b >= 1:
                cps_k = start_row_load(b - 1, keep_off, o_keep, keep_sem)
            cps_s.wait()
            if b >= 2:
                pl.semaphore_wait(credit_sem, 1)
                rdmas[b - 2].wait_send()
            for n in range(N // NT):
                send_buf[slot, :, n * NT:(n + 1) * NT] = jnp.dot(
                    o_send[...], w_ref[:, n * NT:(n + 1) * NT],
                    preferred_element_type=jnp.float32,
                ).astype(jnp.bfloat16)
            rdma = pltpu.make_async_remote_copy(
                src_ref=send_buf.at[slot],
                dst_ref=recv_buf.at[slot],
                send_sem=send_sems.at[slot],
                recv_sem=recv_sems.at[slot],
                device_id=partner,
                device_id_type=pl.DeviceIdType.MESH,
            )
            rdma.start()
            rdmas.append(rdma)

            if b >= 1:
                cps_k.wait()
                consume(b - 1, rdmas[b - 1])

        cps_k = start_row_load(B - 1, keep_off, o_keep, keep_sem)
        cps_k.wait()
        consume(B - 1, rdmas[B - 1])

        rdmas[B - 2].wait_send()
        rdmas[B - 1].wait_send()
        pl.semaphore_wait(credit_sem, 2)
        for st in pending_store:
            if st is not None:
                st.wait()

    return pl.pallas_call(
        body,
        out_shape=jax.ShapeDtypeStruct((B, S_HALF, N), jnp.bfloat16),
        in_specs=[
            pl.BlockSpec(memory_space=pl.ANY),
            pl.BlockSpec(memory_space=pltpu.VMEM),
        ],
        out_specs=pl.BlockSpec(memory_space=pl.ANY),
        scratch_shapes=[
            pltpu.VMEM((S_HALF, K), jnp.bfloat16),
            pltpu.VMEM((S_HALF, K), jnp.bfloat16),
            pltpu.VMEM((2, S_HALF, N), jnp.bfloat16),
            pltpu.VMEM((2, S_HALF, N), jnp.bfloat16),
            pltpu.VMEM((S_HALF, NT), jnp.float32),
            pltpu.VMEM((S_HALF, NT), jnp.bfloat16),
            pltpu.SemaphoreType.DMA,
            pltpu.SemaphoreType.DMA,
            pltpu.SemaphoreType.DMA((2,)),
            pltpu.SemaphoreType.DMA((2,)),
            pltpu.SemaphoreType.DMA((2,)),
            pltpu.SemaphoreType.REGULAR,
        ],
        compiler_params=pltpu.CompilerParams(
            collective_id=0,
            vmem_limit_bytes=50250000,
        ),
    )(o, w)


# device time: 262156 ns/iter; 1.0589x vs baseline; 1.0278x over previous
import jax
import jax.numpy as jnp
from jax import lax
from jax.experimental import pallas as pl
from jax.experimental.pallas import tpu as pltpu

B = 4
S = 1024
H_SHARD = 16
D = 128
K = H_SHARD * D
N = 4096
S_HALF = S // 2
NT = 2048


def kernel(O, Wo):
    o = O.astype(jnp.bfloat16).reshape(B, S, K)
    w = Wo.astype(jnp.bfloat16)

    def body(
        o_hbm, w_ref, out_hbm,
        o_send, o_keep, send_buf, recv_buf, acc_stage, out_stage,
        load_sem, keep_sem, store_sems, send_sems, recv_sems,
        sub_send_sems, sub_recv_sems, credit_sem,
    ):
        my_x = lax.axis_index("x")
        my_y = lax.axis_index("y")
        my_z = lax.axis_index("z")
        partner = (my_x, my_y, 1 - my_z)

        barrier_sem = pltpu.get_barrier_semaphore()
        pl.semaphore_signal(
            barrier_sem, inc=1,
            device_id=partner, device_id_type=pl.DeviceIdType.MESH,
        )
        pl.semaphore_wait(barrier_sem, 1)

        send_off = (1 - my_z) * S_HALF
        keep_off = my_z * S_HALF

        def start_row_load(b, off, dst, sem):
            cp = pltpu.make_async_copy(
                o_hbm.at[b, pl.ds(off, S_HALF), :], dst, sem
            )
            cp.start()
            return cp

        pending_store = [None, None]

        def consume(b, rdma_b):
            slot = b % 2
            for n in range(N // NT):
                s = n % 2
                acc_stage[...] = jnp.dot(
                    o_keep[...], w_ref[:, n * NT:(n + 1) * NT],
                    preferred_element_type=jnp.float32,
                )
                if n == 0:
                    for r in (rdma_b if isinstance(rdma_b, list)
                              else [rdma_b]):
                        r.wait_recv()
                acc_stage[...] = acc_stage[...] + recv_buf[
                    slot, :, n * NT:(n + 1) * NT
                ].astype(jnp.float32)
                if pending_store[0] is not None:
                    pending_store[0].wait()
                out_stage[...] = acc_stage[...].astype(jnp.bfloat16)
                st = pltpu.make_async_copy(
                    out_stage,
                    out_hbm.at[b, :, pl.ds(n * NT, NT)],
                    store_sems.at[s],
                )
                st.start()
                pending_store[0] = st
            pl.semaphore_signal(
                credit_sem, inc=1,
                device_id=partner, device_id_type=pl.DeviceIdType.MESH,
            )

        SUB = 4
        SR = S_HALF // SUB

        rdmas = []
        for b in range(B):
            slot = b % 2

            cps_s = start_row_load(b, send_off, o_send, load_sem)
            if b >= 1:
                cps_k = start_row_load(b - 1, keep_off, o_keep, keep_sem)
            cps_s.wait()
            if b >= 2:
                pl.semaphore_wait(credit_sem, 1)
                for r in rdmas[b - 2] if isinstance(rdmas[b - 2], list) \
                        else [rdmas[b - 2]]:
                    r.wait_send()
            if b == 0:
                subs = []
                for u in range(SUB):
                    for n in range(N // NT):
                        send_buf[0, pl.ds(u * SR, SR),
                                 n * NT:(n + 1) * NT] = jnp.dot(
                            o_send[pl.ds(u * SR, SR), :],
                            w_ref[:, n * NT:(n + 1) * NT],
                            preferred_element_type=jnp.float32,
                        ).astype(jnp.bfloat16)
                    sub = pltpu.make_async_remote_copy(
                        src_ref=send_buf.at[0, pl.ds(u * SR, SR), :],
                        dst_ref=recv_buf.at[0, pl.ds(u * SR, SR), :],
                        send_sem=sub_send_sems.at[u],
                        recv_sem=sub_recv_sems.at[u],
                        device_id=partner,
                        device_id_type=pl.DeviceIdType.MESH,
                    )
                    sub.start()
                    subs.append(sub)
                rdmas.append(subs)
            else:
                for n in range(N // NT):
                    send_buf[slot, :, n * NT:(n + 1) * NT] = jnp.dot(
                        o_send[...], w_ref[:, n * NT:(n + 1) * NT],
                        preferred_element_type=jnp.float32,
                    ).astype(jnp.bfloat16)
                rdma = pltpu.make_async_remote_copy(
                    src_ref=send_buf.at[slot],
                    dst_ref=recv_buf.at[slot],
                    send_sem=send_sems.at[slot],
                    recv_sem=recv_sems.at[slot],
                    device_id=partner,
                    device_id_type=pl.DeviceIdType.MESH,
                )
                rdma.start()
                rdmas.append(rdma)

            if b >= 1:
                cps_k.wait()
                consume(b - 1, rdmas[b - 1])

        cps_k = start_row_load(B - 1, keep_off, o_keep, keep_sem)
        cps_k.wait()
        consume(B - 1, rdmas[B - 1])

        rdmas[B - 2].wait_send()
        rdmas[B - 1].wait_send()
        pl.semaphore_wait(credit_sem, 2)
        for st in pending_store:
            if st is not None:
                st.wait()

    return pl.pallas_call(
        body,
        out_shape=jax.ShapeDtypeStruct((B, S_HALF, N), jnp.bfloat16),
        in_specs=[
            pl.BlockSpec(memory_space=pl.ANY),
            pl.BlockSpec(memory_space=pltpu.VMEM),
        ],
        out_specs=pl.BlockSpec(memory_space=pl.ANY),
        scratch_shapes=[
            pltpu.VMEM((S_HALF, K), jnp.bfloat16),
            pltpu.VMEM((S_HALF, K), jnp.bfloat16),
            pltpu.VMEM((2, S_HALF, N), jnp.bfloat16),
            pltpu.VMEM((2, S_HALF, N), jnp.bfloat16),
            pltpu.VMEM((S_HALF, NT), jnp.float32),
            pltpu.VMEM((S_HALF, NT), jnp.bfloat16),
            pltpu.SemaphoreType.DMA,
            pltpu.SemaphoreType.DMA,
            pltpu.SemaphoreType.DMA((2,)),
            pltpu.SemaphoreType.DMA((2,)),
            pltpu.SemaphoreType.DMA((2,)),
            pltpu.SemaphoreType.DMA((4,)),
            pltpu.SemaphoreType.DMA((4,)),
            pltpu.SemaphoreType.REGULAR,
        ],
        compiler_params=pltpu.CompilerParams(
            collective_id=0,
            vmem_limit_bytes=50250000,
        ),
    )(o, w)
